# baseline (speedup 1.0000x reference)
import functools

import jax
import jax.numpy as jnp
from jax import lax
from jax.experimental import pallas as pl
from jax.experimental.pallas import tpu as pltpu
from jax.experimental.pallas import tpu_sc as plsc

EMBED_DIM = 32
NUM_CORES = 2
NUM_SUBCORES = 16
CHUNK = 128


def _sc_gather_fm(table, idx_t):
    n_fields, batch = idx_t.shape
    n_chunks = batch // CHUNK
    assert n_chunks % 2 == 0
    mesh = plsc.VectorSubcoreMesh(core_axis_name="c", subcore_axis_name="s")

    @functools.partial(
        pl.kernel,
        out_type=jax.ShapeDtypeStruct((n_fields, batch, EMBED_DIM), table.dtype),
        mesh=mesh,
        scratch_types=[
            pltpu.VMEM((2, CHUNK), jnp.int32),
            pltpu.VMEM((CHUNK, EMBED_DIM), jnp.float32),
            pltpu.VMEM((CHUNK, EMBED_DIM), jnp.float32),
            pltpu.SemaphoreType.DMA,
            pltpu.SemaphoreType.DMA,
            pltpu.SemaphoreType.DMA,
            pltpu.SemaphoreType.DMA,
            pltpu.SemaphoreType.DMA,
            pltpu.SemaphoreType.DMA,
        ],
        compiler_params=pltpu.CompilerParams(use_tc_tiling_on_sc=False),
    )
    def gather_kernel(table_hbm, idx_hbm, out_hbm, ibuf, r0, r1,
                      isem0, isem1, gsem0, gsem1, osem0, osem1):
        wid = lax.axis_index("s") * NUM_CORES + lax.axis_index("c")

        @pl.when(wid < n_fields)
        def _():
            f = wid
            row_bufs = (r0, r1)
            isems = (isem0, isem1)
            gsems = (gsem0, gsem1)
            osems = (osem0, osem1)

            pltpu.async_copy(
                idx_hbm.at[f, pl.ds(0, CHUNK)], ibuf.at[0], isem0)
            pltpu.async_copy(
                idx_hbm.at[f, pl.ds(CHUNK, CHUNK)], ibuf.at[1], isem1)

            @pl.loop(0, n_chunks // 2)
            def _(u):
                for b in range(2):
                    rb = row_bufs[b]
                    s = 2 * u + b
                    off = s * CHUNK

                    @pl.when(u >= 1)
                    def _():
                        pltpu.make_async_copy(
                            out_hbm.at[f, pl.ds(0, CHUNK)], rb,
                            osems[b]).wait()

                    pltpu.make_async_copy(
                        idx_hbm.at[f, pl.ds(0, CHUNK)], ibuf.at[b],
                        isems[b]).wait()

                    pltpu.async_copy(
                        table_hbm.at[ibuf.at[b]], rb, gsems[b])

                    pltpu.make_async_copy(
                        out_hbm.at[f, pl.ds(0, CHUNK)], rb, gsems[b]).wait()
                    pltpu.async_copy(
                        rb, out_hbm.at[f, pl.ds(off, CHUNK)], osems[b])

                    @pl.when(u < n_chunks // 2 - 1)
                    def _():
                        pltpu.async_copy(
                            idx_hbm.at[f, pl.ds(off + 2 * CHUNK, CHUNK)],
                            ibuf.at[b], isems[b])

            pltpu.make_async_copy(
                out_hbm.at[f, pl.ds(0, CHUNK)], r0, osem0).wait()
            pltpu.make_async_copy(
                out_hbm.at[f, pl.ds(0, CHUNK)], r1, osem1).wait()

    return gather_kernel(table, idx_t)


@jax.jit
def kernel(inputs, indices):
    idx_t = indices.T
    out_fm = _sc_gather_fm(inputs, idx_t)
    return jnp.swapaxes(out_fm, 0, 1)


# speedup vs baseline: 1.0248x; 1.0248x over previous
import functools

import jax
import jax.numpy as jnp
from jax import lax
from jax.experimental import pallas as pl
from jax.experimental.pallas import tpu as pltpu
from jax.experimental.pallas import tpu_sc as plsc

EMBED_DIM = 32
NUM_CORES = 2
NUM_SUBCORES = 16
NUM_WORKERS = NUM_CORES * NUM_SUBCORES
RPS = 32


def _sc_gather(table, indices):
    batch, n_fields = indices.shape
    rows_per_worker = batch // NUM_WORKERS
    n_super = rows_per_worker // RPS
    assert n_super % 2 == 0
    mesh = plsc.VectorSubcoreMesh(core_axis_name="c", subcore_axis_name="s")

    @functools.partial(
        pl.kernel,
        out_type=jax.ShapeDtypeStruct((batch, n_fields, EMBED_DIM), table.dtype),
        mesh=mesh,
        scratch_types=[
            pltpu.VMEM((RPS, n_fields), jnp.int32),
            pltpu.VMEM((RPS, n_fields), jnp.int32),
            pltpu.VMEM((RPS, n_fields, EMBED_DIM), jnp.float32),
            pltpu.VMEM((RPS, n_fields, EMBED_DIM), jnp.float32),
            pltpu.SemaphoreType.DMA,
            pltpu.SemaphoreType.DMA,
            pltpu.SemaphoreType.DMA,
            pltpu.SemaphoreType.DMA,
            pltpu.SemaphoreType.DMA,
            pltpu.SemaphoreType.DMA,
        ],
        compiler_params=pltpu.CompilerParams(use_tc_tiling_on_sc=False),
    )
    def gather_kernel(table_hbm, idx_hbm, out_hbm, i0, i1, r0, r1,
                      isem0, isem1, gsem0, gsem1, osem0, osem1):
        wid = lax.axis_index("s") * NUM_CORES + lax.axis_index("c")
        base = wid * rows_per_worker
        idx_bufs = (i0, i1)
        row_bufs = (r0, r1)
        isems = (isem0, isem1)
        gsems = (gsem0, gsem1)
        osems = (osem0, osem1)

        pltpu.async_copy(idx_hbm.at[pl.ds(base, RPS)], i0, isem0)
        pltpu.async_copy(idx_hbm.at[pl.ds(base + RPS, RPS)], i1, isem1)

        @pl.loop(0, n_super // 2)
        def _(u):
            for b in range(2):
                ib, rb = idx_bufs[b], row_bufs[b]
                s = 2 * u + b
                off = base + s * RPS

                @pl.when(u >= 1)
                def _():
                    pltpu.make_async_copy(
                        out_hbm.at[pl.ds(base, RPS)], rb, osems[b]).wait()

                pltpu.make_async_copy(
                    idx_hbm.at[pl.ds(base, RPS)], ib, isems[b]).wait()

                for j in range(RPS):
                    pltpu.async_copy(
                        table_hbm.at[ib.at[j]], rb.at[j], gsems[b])

                pltpu.make_async_copy(
                    out_hbm.at[pl.ds(base, RPS)], rb, gsems[b]).wait()
                pltpu.async_copy(rb, out_hbm.at[pl.ds(off, RPS)], osems[b])

                @pl.when(u < n_super // 2 - 1)
                def _():
                    pltpu.async_copy(
                        idx_hbm.at[pl.ds(base + (s + 2) * RPS, RPS)],
                        ib, isems[b])

        pltpu.make_async_copy(
            out_hbm.at[pl.ds(base, RPS)], r0, osem0).wait()
        pltpu.make_async_copy(
            out_hbm.at[pl.ds(base, RPS)], r1, osem1).wait()

    return gather_kernel(table, indices)


@jax.jit
def kernel(inputs, indices):
    return _sc_gather(inputs, indices)
